# baseline (speedup 1.0000x reference)
import functools

import jax
import jax.numpy as jnp
from jax import lax
from jax.experimental import pallas as pl
from jax.experimental.pallas import tpu as pltpu
from jax.experimental.pallas import tpu_sc as plsc

S, B, H = 2048, 1, 768
F = 3072
E = 8
TOPK = 2
T = S * B
A = T * TOPK
TM = 128
NT = A // TM + E
NP = NT * TM

NC, NS, L = 2, 16, 16
NW = NC * NS
GROWS = NP // NW
GCH = 80
CTOK = T // NW

def _sc_gather_body(idx0_hbm, idx1_hbm, x_hbm, out_hbm, j0_v, j1_v, xb_v,
                    s0, s1):
    wid = lax.axis_index("s") * NC + lax.axis_index("c")
    base = wid * CTOK
    pltpu.sync_copy(idx0_hbm.at[pl.ds(base, CTOK)], j0_v)
    pltpu.sync_copy(idx1_hbm.at[pl.ds(base, CTOK)], j1_v)
    pltpu.sync_copy(x_hbm.at[pl.ds(base, CTOK)], xb_v)
    c0 = pltpu.async_copy(xb_v, out_hbm.at[j0_v], s0)
    c1 = pltpu.async_copy(xb_v, out_hbm.at[j1_v], s1)
    c0.wait()
    c1.wait()


def _gmm_body(meta_ref, xs_ref, w1_ref, b1_ref, w2_ref, b2_ref, out_ref):
    i = pl.program_id(0)

    @pl.when(i < meta_ref[NT])
    def _():
        h1 = jnp.dot(xs_ref[...], w1_ref[0],
                     preferred_element_type=jnp.float32) + b1_ref[0]
        h1 = jnp.maximum(h1, 0.0)
        out_ref[...] = jnp.dot(h1, w2_ref[0],
                               preferred_element_type=jnp.float32) + b2_ref[0]


_gmm = pl.pallas_call(
    _gmm_body,
    grid_spec=pltpu.PrefetchScalarGridSpec(
        num_scalar_prefetch=1,
        grid=(NT,),
        in_specs=[
            pl.BlockSpec((TM, H), lambda i, m: (i, 0)),
            pl.BlockSpec((1, H, F), lambda i, m: (m[i], 0, 0)),
            pl.BlockSpec((1, 1, F), lambda i, m: (m[i], 0, 0)),
            pl.BlockSpec((1, F, H), lambda i, m: (m[i], 0, 0)),
            pl.BlockSpec((1, 1, H), lambda i, m: (m[i], 0, 0)),
        ],
        out_specs=pl.BlockSpec((TM, H), lambda i, m: (i, 0)),
    ),
    out_shape=jax.ShapeDtypeStruct((NP, H), jnp.float32),
)


def _sc_combine_body(idx0_hbm, idx1_hbm, p_hbm, rows_hbm, y_hbm,
                     i0_v, i1_v, p_v, b0_v, b1_v, s0, s1):
    wid = lax.axis_index("s") * NC + lax.axis_index("c")
    base = wid * CTOK
    pltpu.sync_copy(idx0_hbm.at[pl.ds(base, CTOK)], i0_v)
    pltpu.sync_copy(idx1_hbm.at[pl.ds(base, CTOK)], i1_v)
    pltpu.sync_copy(p_hbm.at[pl.ds(base, CTOK)], p_v)
    c0 = pltpu.async_copy(rows_hbm.at[i0_v], b0_v, s0)
    c1 = pltpu.async_copy(rows_hbm.at[i1_v], b1_v, s1)
    c0.wait()
    c1.wait()

    def row_body(r, _):
        p0 = p_v[r, 0, :]
        p1 = p_v[r, 1, :]
        for o in range(H // L):
            sl = pl.ds(o * L, L)
            b0_v[r, sl] = p0 * b0_v[r, sl] + p1 * b1_v[r, sl]
        return 0
    lax.fori_loop(0, CTOK, row_body, 0)
    pltpu.sync_copy(b0_v, y_hbm.at[pl.ds(base, CTOK)])


@functools.lru_cache(maxsize=1)
def _sc_kernels():
    mesh = plsc.VectorSubcoreMesh(core_axis_name="c", subcore_axis_name="s")
    sc_gather = pl.kernel(
        _sc_gather_body,
        mesh=mesh,
        compiler_params=pltpu.CompilerParams(
            needs_layout_passes=False, use_tc_tiling_on_sc=True),
        out_type=jax.ShapeDtypeStruct((NP, H), jnp.float32),
        scratch_types=[
            pltpu.VMEM((CTOK,), jnp.int32),
            pltpu.VMEM((CTOK,), jnp.int32),
            pltpu.VMEM((CTOK, H), jnp.float32),
            pltpu.SemaphoreType.DMA,
            pltpu.SemaphoreType.DMA,
        ],
    )
    sc_combine = pl.kernel(
        _sc_combine_body,
        mesh=mesh,
        compiler_params=pltpu.CompilerParams(use_tc_tiling_on_sc=True),
        out_type=jax.ShapeDtypeStruct((T, H), jnp.float32),
        scratch_types=[
            pltpu.VMEM((CTOK,), jnp.int32),
            pltpu.VMEM((CTOK,), jnp.int32),
            pltpu.VMEM((CTOK, TOPK, L), jnp.float32),
            pltpu.VMEM((CTOK, H), jnp.float32),
            pltpu.VMEM((CTOK, H), jnp.float32),
            pltpu.SemaphoreType.DMA,
            pltpu.SemaphoreType.DMA,
        ],
    )
    return sc_gather, sc_combine


def kernel(x, Wg, W1, b1, W2, b2):
    sc_gather, sc_combine = _sc_kernels()
    x_flat = x.reshape(T, H)

    logits = x_flat @ Wg
    probs = jax.nn.softmax(logits, axis=-1)
    topk_probs, topk_idx = jax.lax.top_k(probs, TOPK)

    e_flat = topk_idx.reshape(A)
    onehot = (e_flat[:, None] == jnp.arange(E, dtype=e_flat.dtype)[None, :]
              ).astype(jnp.int32)
    ranks = jnp.cumsum(onehot, axis=0) - onehot
    counts = jnp.sum(onehot, axis=0)
    tiles_per_e = (counts + TM - 1) // TM
    pad_off = jnp.concatenate(
        [jnp.zeros((1,), jnp.int32),
         jnp.cumsum(tiles_per_e * TM).astype(jnp.int32)[:-1]])
    ppos = (pad_off[e_flat] +
            jnp.take_along_axis(ranks, e_flat[:, None], axis=1)[:, 0]
            ).astype(jnp.int32)

    tile_off = jnp.cumsum(tiles_per_e).astype(jnp.int32)
    tile_expert = jnp.minimum(
        jnp.sum((jnp.arange(NT, dtype=jnp.int32)[:, None] >=
                 tile_off[None, :]).astype(jnp.int32), axis=1),
        E - 1).astype(jnp.int32)
    nused = tile_off[E - 1]
    meta = jnp.concatenate([tile_expert, nused[None]])

    ppos2 = ppos.reshape(T, TOPK)
    idx0 = ppos2[:, 0]
    idx1 = ppos2[:, 1]
    xs_pad = sc_gather(idx0, idx1, x_flat)
    out_pad = _gmm(meta, xs_pad, W1, b1.reshape(E, 1, F), W2,
                   b2.reshape(E, 1, H))

    p_bc = jnp.broadcast_to(topk_probs[:, :, None], (T, TOPK, L))
    y_flat = sc_combine(idx0, idx1, p_bc, out_pad)
    return y_flat.reshape(S, B, H)


# speedup vs baseline: 1.0072x; 1.0072x over previous
import functools

import jax
import jax.numpy as jnp
from jax import lax
from jax.experimental import pallas as pl
from jax.experimental.pallas import tpu as pltpu
from jax.experimental.pallas import tpu_sc as plsc

S, B, H = 2048, 1, 768
F = 3072
E = 8
TOPK = 2
T = S * B
A = T * TOPK
TM = 128
NT = A // TM + E
NP = NT * TM

NC, NS, L = 2, 16, 16
NW = NC * NS
GROWS = NP // NW
GCH = 80
CTOK = T // NW

def _sc_gather_body(idx0_hbm, idx1_hbm, p_hbm, x_hbm, out_hbm, j0_v, j1_v,
                    xb_v, s0, s1):
    del p_hbm
    wid = lax.axis_index("s") * NC + lax.axis_index("c")
    base = wid * CTOK
    pltpu.sync_copy(idx0_hbm.at[pl.ds(base, CTOK)], j0_v)
    pltpu.sync_copy(idx1_hbm.at[pl.ds(base, CTOK)], j1_v)
    pltpu.sync_copy(x_hbm.at[pl.ds(base, CTOK)], xb_v)
    c0 = pltpu.async_copy(xb_v, out_hbm.at[j0_v], s0)
    c1 = pltpu.async_copy(xb_v, out_hbm.at[j1_v], s1)
    c0.wait()
    c1.wait()


def _gmm_body(meta_ref, xs_ref, w1_ref, b1_ref, w2_ref, b2_ref, out_ref):
    i = pl.program_id(0)

    @pl.when(i < meta_ref[NT])
    def _():
        h1 = jnp.dot(xs_ref[...], w1_ref[0],
                     preferred_element_type=jnp.float32) + b1_ref[0]
        h1 = jnp.maximum(h1, 0.0)
        out_ref[...] = jnp.dot(h1, w2_ref[0],
                               preferred_element_type=jnp.float32) + b2_ref[0]


_gmm = pl.pallas_call(
    _gmm_body,
    grid_spec=pltpu.PrefetchScalarGridSpec(
        num_scalar_prefetch=1,
        grid=(NT,),
        in_specs=[
            pl.BlockSpec((TM, H), lambda i, m: (i, 0)),
            pl.BlockSpec((1, H, F), lambda i, m: (m[i], 0, 0)),
            pl.BlockSpec((1, 1, F), lambda i, m: (m[i], 0, 0)),
            pl.BlockSpec((1, F, H), lambda i, m: (m[i], 0, 0)),
            pl.BlockSpec((1, 1, H), lambda i, m: (m[i], 0, 0)),
        ],
        out_specs=pl.BlockSpec((TM, H), lambda i, m: (i, 0)),
    ),
    out_shape=jax.ShapeDtypeStruct((NP, H), jnp.float32),
)


def _sc_combine_body(idx0_hbm, idx1_hbm, p_hbm, rows_hbm, y_hbm,
                     i0_v, i1_v, p_v, b0_v, b1_v, s0, s1):
    wid = lax.axis_index("s") * NC + lax.axis_index("c")
    base = wid * CTOK
    pltpu.sync_copy(idx0_hbm.at[pl.ds(base, CTOK)], i0_v)
    pltpu.sync_copy(idx1_hbm.at[pl.ds(base, CTOK)], i1_v)
    pltpu.sync_copy(p_hbm.at[pl.ds(base, CTOK)], p_v)
    c0 = pltpu.async_copy(rows_hbm.at[i0_v], b0_v, s0)
    c1 = pltpu.async_copy(rows_hbm.at[i1_v], b1_v, s1)
    c0.wait()
    c1.wait()

    def row_body(r, _):
        p0 = p_v[r, 0, :]
        p1 = p_v[r, 1, :]
        for o in range(H // L):
            sl = pl.ds(o * L, L)
            b0_v[r, sl] = p0 * b0_v[r, sl] + p1 * b1_v[r, sl]
        return 0
    lax.fori_loop(0, CTOK, row_body, 0)
    pltpu.sync_copy(b0_v, y_hbm.at[pl.ds(base, CTOK)])


@functools.lru_cache(maxsize=1)
def _sc_kernels():
    mesh = plsc.VectorSubcoreMesh(core_axis_name="c", subcore_axis_name="s")
    sc_gather = pl.kernel(
        _sc_gather_body,
        mesh=mesh,
        compiler_params=pltpu.CompilerParams(
            needs_layout_passes=False, use_tc_tiling_on_sc=True),
        out_type=jax.ShapeDtypeStruct((NP, H), jnp.float32),
        scratch_types=[
            pltpu.VMEM((CTOK,), jnp.int32),
            pltpu.VMEM((CTOK,), jnp.int32),
            pltpu.VMEM((CTOK, H), jnp.float32),
            pltpu.SemaphoreType.DMA,
            pltpu.SemaphoreType.DMA,
        ],
    )
    sc_combine = pl.kernel(
        _sc_combine_body,
        mesh=mesh,
        compiler_params=pltpu.CompilerParams(use_tc_tiling_on_sc=True),
        out_type=jax.ShapeDtypeStruct((T, H), jnp.float32),
        scratch_types=[
            pltpu.VMEM((CTOK,), jnp.int32),
            pltpu.VMEM((CTOK,), jnp.int32),
            pltpu.VMEM((CTOK, TOPK, L), jnp.float32),
            pltpu.VMEM((CTOK, H), jnp.float32),
            pltpu.VMEM((CTOK, H), jnp.float32),
            pltpu.SemaphoreType.DMA,
            pltpu.SemaphoreType.DMA,
        ],
    )
    return sc_gather, sc_combine


def kernel(x, Wg, W1, b1, W2, b2):
    sc_gather, sc_combine = _sc_kernels()
    x_flat = x.reshape(T, H)

    logits = x_flat @ Wg
    probs = jax.nn.softmax(logits, axis=-1)
    topk_probs, topk_idx = jax.lax.top_k(probs, TOPK)

    e_flat = topk_idx.reshape(A)
    onehot = (e_flat[:, None] == jnp.arange(E, dtype=e_flat.dtype)[None, :]
              ).astype(jnp.int32)
    ranks = jnp.cumsum(onehot, axis=0) - onehot
    counts = jnp.sum(onehot, axis=0)
    tiles_per_e = (counts + TM - 1) // TM
    pad_off = jnp.concatenate(
        [jnp.zeros((1,), jnp.int32),
         jnp.cumsum(tiles_per_e * TM).astype(jnp.int32)[:-1]])
    ppos = (pad_off[e_flat] +
            jnp.take_along_axis(ranks, e_flat[:, None], axis=1)[:, 0]
            ).astype(jnp.int32)

    tile_off = jnp.cumsum(tiles_per_e).astype(jnp.int32)
    tile_expert = jnp.minimum(
        jnp.sum((jnp.arange(NT, dtype=jnp.int32)[:, None] >=
                 tile_off[None, :]).astype(jnp.int32), axis=1),
        E - 1).astype(jnp.int32)
    nused = tile_off[E - 1]
    meta = jnp.concatenate([tile_expert, nused[None]])

    ppos2 = ppos.reshape(T, TOPK)
    idx0 = ppos2[:, 0]
    idx1 = ppos2[:, 1]
    p_bc = jnp.broadcast_to(topk_probs[:, :, None], (T, TOPK, L))
    xs_pad = sc_gather(idx0, idx1, p_bc, x_flat)
    out_pad = _gmm(meta, xs_pad, W1, b1.reshape(E, 1, F), W2,
                   b2.reshape(E, 1, H))

    y_flat = sc_combine(idx0, idx1, p_bc, out_pad)
    return y_flat.reshape(S, B, H)


# speedup vs baseline: 1.0151x; 1.0078x over previous
import functools

import jax
import jax.numpy as jnp
from jax import lax
from jax.experimental import pallas as pl
from jax.experimental.pallas import tpu as pltpu
from jax.experimental.pallas import tpu_sc as plsc

S, B, H = 2048, 1, 768
F = 3072
E = 8
TOPK = 2
T = S * B
A = T * TOPK
TM = 128
NT = A // TM + E
NP = NT * TM

NC, NS, L = 2, 16, 16
NW = NC * NS
GROWS = NP // NW
GCH = 80
CTOK = T // NW

def _sc_gather_body(idx0_hbm, idx1_hbm, p_hbm, x_hbm, out_hbm, j0_v, j1_v,
                    xb_v, s0, s1):
    del p_hbm
    wid = lax.axis_index("s") * NC + lax.axis_index("c")
    base = wid * CTOK
    pltpu.sync_copy(idx0_hbm.at[pl.ds(base, CTOK)], j0_v)
    pltpu.sync_copy(idx1_hbm.at[pl.ds(base, CTOK)], j1_v)
    pltpu.sync_copy(x_hbm.at[pl.ds(base, CTOK)], xb_v)
    c0 = pltpu.async_copy(xb_v, out_hbm.at[j0_v], s0)
    c1 = pltpu.async_copy(xb_v, out_hbm.at[j1_v], s1)
    c0.wait()
    c1.wait()


def _gmm_body(meta_ref, xs_ref, w1_ref, b1_ref, w2_ref, b2_ref, out_ref):
    i = pl.program_id(0)

    @pl.when(i < meta_ref[NT])
    def _():
        h1 = jnp.dot(xs_ref[...], w1_ref[0],
                     preferred_element_type=jnp.float32) + b1_ref[0]
        h1 = jnp.maximum(h1, 0.0)
        out_ref[...] = jnp.dot(h1, w2_ref[0],
                               preferred_element_type=jnp.float32) + b2_ref[0]


_gmm = pl.pallas_call(
    _gmm_body,
    grid_spec=pltpu.PrefetchScalarGridSpec(
        num_scalar_prefetch=1,
        grid=(NT,),
        in_specs=[
            pl.BlockSpec((TM, H), lambda i, m: (i, 0)),
            pl.BlockSpec((1, H, F), lambda i, m: (m[i], 0, 0)),
            pl.BlockSpec((1, 1, F), lambda i, m: (m[i], 0, 0)),
            pl.BlockSpec((1, F, H), lambda i, m: (m[i], 0, 0)),
            pl.BlockSpec((1, 1, H), lambda i, m: (m[i], 0, 0)),
        ],
        out_specs=pl.BlockSpec((TM, H), lambda i, m: (i, 0)),
    ),
    out_shape=jax.ShapeDtypeStruct((NP, H), jnp.float32),
)


HTOK = CTOK // 2


def _sc_combine_body(idx0_hbm, idx1_hbm, p_hbm, rows_hbm, y_hbm,
                     i0a_v, i1a_v, i0b_v, i1b_v, p_v,
                     b0a_v, b1a_v, b0b_v, b1b_v, s0, s1):
    wid = lax.axis_index("s") * NC + lax.axis_index("c")
    base = wid * CTOK
    pltpu.sync_copy(idx0_hbm.at[pl.ds(base, HTOK)], i0a_v)
    pltpu.sync_copy(idx1_hbm.at[pl.ds(base, HTOK)], i1a_v)
    c0a = pltpu.async_copy(rows_hbm.at[i0a_v], b0a_v, s0)
    c1a = pltpu.async_copy(rows_hbm.at[i1a_v], b1a_v, s0)
    pltpu.sync_copy(idx0_hbm.at[pl.ds(base + HTOK, HTOK)], i0b_v)
    pltpu.sync_copy(idx1_hbm.at[pl.ds(base + HTOK, HTOK)], i1b_v)
    c0b = pltpu.async_copy(rows_hbm.at[i0b_v], b0b_v, s1)
    c1b = pltpu.async_copy(rows_hbm.at[i1b_v], b1b_v, s1)
    pltpu.sync_copy(p_hbm.at[pl.ds(base, CTOK)], p_v)

    def make_body(b0, b1, poff):
        def row_body(r, _):
            p0 = p_v[poff + r, 0, :]
            p1 = p_v[poff + r, 1, :]
            for o in range(H // L):
                sl = pl.ds(o * L, L)
                b0[r, sl] = p0 * b0[r, sl] + p1 * b1[r, sl]
            return 0
        return row_body

    c0a.wait()
    c1a.wait()
    lax.fori_loop(0, HTOK, make_body(b0a_v, b1a_v, 0), 0)
    ya = pltpu.async_copy(b0a_v, y_hbm.at[pl.ds(base, HTOK)], s0)
    c0b.wait()
    c1b.wait()
    lax.fori_loop(0, HTOK, make_body(b0b_v, b1b_v, HTOK), 0)
    ya.wait()
    pltpu.sync_copy(b0b_v, y_hbm.at[pl.ds(base + HTOK, HTOK)])


@functools.lru_cache(maxsize=1)
def _sc_kernels():
    mesh = plsc.VectorSubcoreMesh(core_axis_name="c", subcore_axis_name="s")
    sc_gather = pl.kernel(
        _sc_gather_body,
        mesh=mesh,
        compiler_params=pltpu.CompilerParams(
            needs_layout_passes=False, use_tc_tiling_on_sc=True),
        out_type=jax.ShapeDtypeStruct((NP, H), jnp.float32),
        scratch_types=[
            pltpu.VMEM((CTOK,), jnp.int32),
            pltpu.VMEM((CTOK,), jnp.int32),
            pltpu.VMEM((CTOK, H), jnp.float32),
            pltpu.SemaphoreType.DMA,
            pltpu.SemaphoreType.DMA,
        ],
    )
    sc_combine = pl.kernel(
        _sc_combine_body,
        mesh=mesh,
        compiler_params=pltpu.CompilerParams(use_tc_tiling_on_sc=True),
        out_type=jax.ShapeDtypeStruct((T, H), jnp.float32),
        scratch_types=[
            pltpu.VMEM((HTOK,), jnp.int32),
            pltpu.VMEM((HTOK,), jnp.int32),
            pltpu.VMEM((HTOK,), jnp.int32),
            pltpu.VMEM((HTOK,), jnp.int32),
            pltpu.VMEM((CTOK, TOPK, L), jnp.float32),
            pltpu.VMEM((HTOK, H), jnp.float32),
            pltpu.VMEM((HTOK, H), jnp.float32),
            pltpu.VMEM((HTOK, H), jnp.float32),
            pltpu.VMEM((HTOK, H), jnp.float32),
            pltpu.SemaphoreType.DMA,
            pltpu.SemaphoreType.DMA,
        ],
    )
    return sc_gather, sc_combine


def kernel(x, Wg, W1, b1, W2, b2):
    sc_gather, sc_combine = _sc_kernels()
    x_flat = x.reshape(T, H)

    logits = x_flat @ Wg
    probs = jax.nn.softmax(logits, axis=-1)
    topk_probs, topk_idx = jax.lax.top_k(probs, TOPK)

    e_flat = topk_idx.reshape(A)
    onehot = (e_flat[:, None] == jnp.arange(E, dtype=e_flat.dtype)[None, :]
              ).astype(jnp.int32)
    ranks = jnp.cumsum(onehot, axis=0) - onehot
    counts = jnp.sum(onehot, axis=0)
    tiles_per_e = (counts + TM - 1) // TM
    pad_off = jnp.concatenate(
        [jnp.zeros((1,), jnp.int32),
         jnp.cumsum(tiles_per_e * TM).astype(jnp.int32)[:-1]])
    ppos = (pad_off[e_flat] +
            jnp.take_along_axis(ranks, e_flat[:, None], axis=1)[:, 0]
            ).astype(jnp.int32)

    tile_off = jnp.cumsum(tiles_per_e).astype(jnp.int32)
    tile_expert = jnp.minimum(
        jnp.sum((jnp.arange(NT, dtype=jnp.int32)[:, None] >=
                 tile_off[None, :]).astype(jnp.int32), axis=1),
        E - 1).astype(jnp.int32)
    nused = tile_off[E - 1]
    meta = jnp.concatenate([tile_expert, nused[None]])

    ppos2 = ppos.reshape(T, TOPK)
    idx0 = ppos2[:, 0]
    idx1 = ppos2[:, 1]
    p_bc = jnp.broadcast_to(topk_probs[:, :, None], (T, TOPK, L))
    xs_pad = sc_gather(idx0, idx1, p_bc, x_flat)
    out_pad = _gmm(meta, xs_pad, W1, b1.reshape(E, 1, F), W2,
                   b2.reshape(E, 1, H))

    y_flat = sc_combine(idx0, idx1, p_bc, out_pad)
    return y_flat.reshape(S, B, H)
